# Initial kernel scaffold; baseline (speedup 1.0000x reference)
import jax
import jax.numpy as jnp
from jax.experimental import pallas as pl


def kernel(positions, cos_cached, sin_cached):
    raise NotImplementedError("write your pallas kernel here")


# speedup vs baseline: 5.8639x; 5.8639x over previous
import functools

import jax
import jax.numpy as jnp
from jax import lax
from jax.experimental import pallas as pl
from jax.experimental.pallas import tpu as pltpu
from jax.experimental.pallas import tpu_sc as plsc

NC = 2
NS = 16
NW = NC * NS
CHUNK = 128


@functools.lru_cache(maxsize=None)
def _make_gather(N, D):
    b_per_w = N // NW
    nchunks = b_per_w // CHUNK
    mesh = plsc.VectorSubcoreMesh(core_axis_name="c", subcore_axis_name="s")

    @functools.partial(
        pl.kernel,
        mesh=mesh,
        out_type=[
            jax.ShapeDtypeStruct((N, D), jnp.float32),
            jax.ShapeDtypeStruct((N, D), jnp.float32),
        ],
        scratch_types=[
            pltpu.VMEM((nchunks, CHUNK), jnp.int32),
            pltpu.VMEM((CHUNK, D), jnp.float32),
            pltpu.VMEM((CHUNK, D), jnp.float32),
            pltpu.SemaphoreType.DMA,
        ],
    )
    def k(idx_hbm, cos_hbm, sin_hbm, cos_out, sin_out, idx_v, cbuf, sbuf, sem):
        wid = lax.axis_index("s") * NC + lax.axis_index("c")
        base = wid * b_per_w
        pltpu.sync_copy(idx_hbm.at[wid], idx_v)
        for j in range(nchunks):
            row0 = base + j * CHUNK
            c_dma = pltpu.async_copy(cos_hbm.at[idx_v.at[j]], cbuf, sem)
            s_dma = pltpu.async_copy(sin_hbm.at[idx_v.at[j]], sbuf, sem)
            c_dma.wait()
            s_dma.wait()
            pltpu.sync_copy(cbuf, cos_out.at[pl.ds(row0, CHUNK)])
            pltpu.sync_copy(sbuf, sin_out.at[pl.ds(row0, CHUNK)])

    return k


def kernel(positions, cos_cached, sin_cached):
    B, S = positions.shape
    V, D = cos_cached.shape
    N = B * S
    idx = positions.astype(jnp.int32).reshape(NW, N // (NW * CHUNK), CHUNK)
    cos, sin = _make_gather(N, D)(idx, cos_cached, sin_cached)
    return cos.reshape(B, S, D), sin.reshape(B, S, D)


# speedup vs baseline: 6.4320x; 1.0969x over previous
import functools

import jax
import jax.numpy as jnp
from jax import lax
from jax.experimental import pallas as pl
from jax.experimental.pallas import tpu as pltpu
from jax.experimental.pallas import tpu_sc as plsc

NC = 2
NS = 16
NW = NC * NS
CHUNK = 128


@functools.lru_cache(maxsize=None)
def _make_gather(N, D):
    b_per_w = N // NW
    nchunks = b_per_w // CHUNK
    mesh = plsc.VectorSubcoreMesh(core_axis_name="c", subcore_axis_name="s")

    @functools.partial(
        pl.kernel,
        mesh=mesh,
        out_type=[
            jax.ShapeDtypeStruct((N, D), jnp.float32),
            jax.ShapeDtypeStruct((N, D), jnp.float32),
        ],
        scratch_types=[
            pltpu.VMEM((nchunks, CHUNK), jnp.int32),
            pltpu.VMEM((2, CHUNK, D), jnp.float32),
            pltpu.VMEM((2, CHUNK, D), jnp.float32),
            pltpu.SemaphoreType.DMA,
            pltpu.SemaphoreType.DMA,
            pltpu.SemaphoreType.DMA,
            pltpu.SemaphoreType.DMA,
        ],
    )
    def k(idx_hbm, cos_hbm, sin_hbm, cos_out, sin_out, idx_v, cbuf, sbuf,
          gsem0, gsem1, ssem0, ssem1):
        wid = lax.axis_index("s") * NC + lax.axis_index("c")
        base = wid * b_per_w
        gsems = (gsem0, gsem1)
        ssems = (ssem0, ssem1)
        pltpu.sync_copy(idx_hbm.at[wid], idx_v)

        def fire_gather(j):
            s = j % 2
            return (
                pltpu.async_copy(cos_hbm.at[idx_v.at[j]], cbuf.at[s], gsems[s]),
                pltpu.async_copy(sin_hbm.at[idx_v.at[j]], sbuf.at[s], gsems[s]),
            )

        def fire_scatter(j):
            s = j % 2
            row0 = base + j * CHUNK
            return (
                pltpu.async_copy(cbuf.at[s], cos_out.at[pl.ds(row0, CHUNK)], ssems[s]),
                pltpu.async_copy(sbuf.at[s], sin_out.at[pl.ds(row0, CHUNK)], ssems[s]),
            )

        gd = fire_gather(0)
        pend = [None, None]
        for j in range(nchunks):
            s = j % 2
            gd[0].wait()
            gd[1].wait()
            pend[s] = fire_scatter(j)
            if j + 1 < nchunks:
                o = 1 - s
                if pend[o] is not None:
                    pend[o][0].wait()
                    pend[o][1].wait()
                    pend[o] = None
                gd = fire_gather(j + 1)
        for p in pend:
            if p is not None:
                p[0].wait()
                p[1].wait()

    return k


def kernel(positions, cos_cached, sin_cached):
    B, S = positions.shape
    V, D = cos_cached.shape
    N = B * S
    idx = positions.astype(jnp.int32).reshape(NW, N // (NW * CHUNK), CHUNK)
    cos, sin = _make_gather(N, D)(idx, cos_cached, sin_cached)
    return cos.reshape(B, S, D), sin.reshape(B, S, D)


# speedup vs baseline: 6.6434x; 1.0329x over previous
import functools

import jax
import jax.numpy as jnp
from jax import lax
from jax.experimental import pallas as pl
from jax.experimental.pallas import tpu as pltpu
from jax.experimental.pallas import tpu_sc as plsc

NC = 2
NS = 16
NW = NC * NS
CHUNK = 128
RING = 3
AHEAD = 2


@functools.lru_cache(maxsize=None)
def _make_gather(N, D):
    b_per_w = N // NW
    nchunks = b_per_w // CHUNK
    mesh = plsc.VectorSubcoreMesh(core_axis_name="c", subcore_axis_name="s")

    @functools.partial(
        pl.kernel,
        mesh=mesh,
        out_type=[
            jax.ShapeDtypeStruct((N, D), jnp.float32),
            jax.ShapeDtypeStruct((N, D), jnp.float32),
        ],
        scratch_types=[
            pltpu.VMEM((nchunks, CHUNK), jnp.int32),
            pltpu.VMEM((RING, CHUNK, D), jnp.float32),
            pltpu.VMEM((RING, CHUNK, D), jnp.float32),
        ] + [pltpu.SemaphoreType.DMA] * (2 * RING),
    )
    def k(idx_hbm, cos_hbm, sin_hbm, cos_out, sin_out, idx_v, cbuf, sbuf,
          *sems):
        wid = lax.axis_index("s") * NC + lax.axis_index("c")
        base = wid * b_per_w
        gsems = sems[:RING]
        ssems = sems[RING:]
        pltpu.sync_copy(idx_hbm.at[wid], idx_v)

        def fire_gather(j):
            s = j % RING
            return (
                pltpu.async_copy(cos_hbm.at[idx_v.at[j]], cbuf.at[s], gsems[s]),
                pltpu.async_copy(sin_hbm.at[idx_v.at[j]], sbuf.at[s], gsems[s]),
            )

        def fire_scatter(j):
            s = j % RING
            row0 = base + j * CHUNK
            return (
                pltpu.async_copy(cbuf.at[s], cos_out.at[pl.ds(row0, CHUNK)], ssems[s]),
                pltpu.async_copy(sbuf.at[s], sin_out.at[pl.ds(row0, CHUNK)], ssems[s]),
            )

        gd = [None] * RING
        pend = [None] * RING
        for j in range(min(AHEAD, nchunks)):
            gd[j % RING] = fire_gather(j)
        for j in range(nchunks):
            s = j % RING
            gd[s][0].wait()
            gd[s][1].wait()
            pend[s] = fire_scatter(j)
            nxt = j + AHEAD
            if nxt < nchunks:
                o = nxt % RING
                if pend[o] is not None:
                    pend[o][0].wait()
                    pend[o][1].wait()
                    pend[o] = None
                gd[o] = fire_gather(nxt)
        for p in pend:
            if p is not None:
                p[0].wait()
                p[1].wait()

    return k


def kernel(positions, cos_cached, sin_cached):
    B, S = positions.shape
    V, D = cos_cached.shape
    N = B * S
    idx = positions.astype(jnp.int32).reshape(NW, N // (NW * CHUNK), CHUNK)
    cos, sin = _make_gather(N, D)(idx, cos_cached, sin_cached)
    return cos.reshape(B, S, D), sin.reshape(B, S, D)
